# baseline (speedup 1.0000x reference)
import functools

import jax
import jax.numpy as jnp
from jax import lax
from jax.experimental import pallas as pl
from jax.experimental.pallas import tpu as pltpu
from jax.experimental.pallas import tpu_sc as plsc

_NC, _NS = 2, 16


def _sc_mesh():
    return plsc.VectorSubcoreMesh(
        core_axis_name="c", subcore_axis_name="s", num_cores=_NC,
        num_subcores=_NS,
    )


@functools.lru_cache(maxsize=None)
def _make_wstack(num_rels: int, num_bases: int, h: int):
    def body(a_ref, v_ref, out_ref):
        r = pl.program_id(0)
        acc = a_ref[r, 0] * v_ref[0]
        for b in range(1, num_bases):
            acc = acc + a_ref[r, b] * v_ref[b]
        out_ref[...] = acc

    return pl.pallas_call(
        body,
        grid=(num_rels,),
        in_specs=[
            pl.BlockSpec((num_rels, num_bases), lambda r: (0, 0)),
            pl.BlockSpec((num_bases, h, h), lambda r: (0, 0, 0)),
        ],
        out_specs=pl.BlockSpec((h, h), lambda r: (0, r)),
        out_shape=jax.ShapeDtypeStruct((h, num_rels * h), jnp.float32),
    )


@functools.lru_cache(maxsize=None)
def _make_matmul(n: int, h: int, num_rels: int, bn: int):
    nb = n // bn

    def body(x_ref, w_ref, out_ref):
        y = jnp.dot(
            x_ref[...].astype(jnp.bfloat16),
            w_ref[...].astype(jnp.bfloat16),
            preferred_element_type=jnp.float32,
        )
        yb = y.astype(jnp.bfloat16)
        for r in range(num_rels):
            out_ref[r] = yb[:, r * h:(r + 1) * h]

    return pl.pallas_call(
        body,
        grid=(nb,),
        in_specs=[
            pl.BlockSpec((bn, h), lambda i: (i, 0)),
            pl.BlockSpec((h, num_rels * h), lambda i: (0, 0)),
        ],
        out_specs=pl.BlockSpec((num_rels, bn, h), lambda i: (0, i, 0)),
        out_shape=jax.ShapeDtypeStruct((num_rels, n, h), jnp.bfloat16),
    )


@functools.lru_cache(maxsize=None)
def _make_finish(n: int, n_pad: int, h: int, bn: int, relu: bool):
    def body(acc_ref, deg_ref, x_ref, loop_ref, b_ref, out_ref):
        acc = acc_ref[0].astype(jnp.float32) + acc_ref[1].astype(jnp.float32)
        deg = deg_ref[0] + deg_ref[1]
        norm = jnp.where(deg > 0.0, 1.0 / jnp.maximum(deg, 1.0), 0.0)
        y = acc * norm + jnp.dot(
            x_ref[...], loop_ref[...], preferred_element_type=jnp.float32
        ) + b_ref[...]
        if relu:
            y = jnp.maximum(y, 0.0)
        out_ref[...] = y

    return pl.pallas_call(
        body,
        grid=(n // bn,),
        in_specs=[
            pl.BlockSpec((2, bn, h), lambda i: (0, i, 0)),
            pl.BlockSpec((2, bn, 1), lambda i: (0, i, 0)),
            pl.BlockSpec((bn, h), lambda i: (i, 0)),
            pl.BlockSpec((h, h), lambda i: (0, 0)),
            pl.BlockSpec((1, h), lambda i: (0, 0)),
        ],
        out_specs=pl.BlockSpec((bn, h), lambda i: (i, 0)),
        out_shape=jax.ShapeDtypeStruct((n, h), jnp.float32),
    )


@functools.lru_cache(maxsize=None)
def _make_sc_deg(n_pad: int, rows_per_tile: int):
    nrows = n_pad // 128
    n_writers = nrows // 8

    def body(dst_hbm, deg_out, dstb, degl, iotab, deg_sh):
        cid = lax.axis_index("c")
        sid = lax.axis_index("s")
        wid = cid * _NS + sid

        zeros16 = jnp.zeros((16,), jnp.float32)
        ones16 = jnp.ones((16,), jnp.float32)
        iota16 = lax.iota(jnp.int32, 16)

        def fill_body(i, _):
            for j in range(128 // 16):
                degl[i, pl.ds(j * 16, 16)] = zeros16
            return 0

        lax.fori_loop(0, nrows, fill_body, 0)
        for j in range(nrows // 16):
            iotab[pl.ds(j * 16, 16)] = iota16 + j * 16

        @pl.when(sid < n_writers)
        def _():
            pltpu.sync_copy(
                degl.at[pl.ds(0, 8)], deg_sh.at[pl.ds(sid * 8, 8)]
            )
        plsc.subcore_barrier()

        pltpu.sync_copy(dst_hbm.at[pl.ds(wid * rows_per_tile, rows_per_tile)],
                        dstb)

        def chunk_body(c, _):
            for j in range(128 // 16):
                dvec = dstb[c, pl.ds(j * 16, 16)]
                hi = lax.shift_right_logical(dvec, 7)
                lo = lax.bitwise_and(dvec, 127)
                plsc.addupdate_scatter(degl, [hi, lo], ones16)
            return 0

        lax.fori_loop(0, rows_per_tile, chunk_body, 0)

        pltpu.sync_copy(degl, deg_sh.at[iotab], add=True)
        plsc.subcore_barrier()

        @pl.when(sid < n_writers)
        def _():
            pltpu.sync_copy(
                deg_sh.at[pl.ds(sid * 8, 8)],
                deg_out.at[cid, pl.ds(sid * 8, 8)],
            )

    return pl.kernel(
        body,
        mesh=_sc_mesh(),
        compiler_params=pltpu.CompilerParams(needs_layout_passes=False, use_tc_tiling_on_sc=False),
        out_type=[jax.ShapeDtypeStruct((_NC, nrows, 128), jnp.float32)],
        scratch_types=[
            pltpu.VMEM((rows_per_tile, 128), jnp.int32),
            pltpu.VMEM((nrows, 128), jnp.float32),
            pltpu.VMEM((nrows,), jnp.int32),
            pltpu.VMEM_SHARED((nrows, 128), jnp.float32),
        ],
    )


@functools.lru_cache(maxsize=None)
def _make_sc_agg(n_pad: int, rows_per_tile: int, h: int, n_nodes: int):
    rows_per_sub = n_pad // _NS
    n128 = rows_per_sub // 128
    n_stages = 2
    stage_rows = rows_per_tile // n_stages

    def body(src_hbm, et_hbm, dst_hbm, t_hbm, acc_out,
             dstb, gidxb, rows0, rows1, acc_sh,
             gsem0, gsem1, ssem0, ssem1):
        cid = lax.axis_index("c")
        sid = lax.axis_index("s")
        wid = cid * _NS + sid
        base_row = wid * rows_per_tile

        zeros16 = jnp.zeros((16,), jnp.float32)

        zeros32 = jnp.zeros((32,), jnp.bfloat16)

        def fill_body(i, _):
            for j in range(h // 32):
                rows0[i, pl.ds(j * 32, 32)] = zeros32
            return 0

        lax.fori_loop(0, 128, fill_body, 0)
        for k in range(n128):
            pltpu.sync_copy(
                rows0, acc_sh.at[pl.ds(sid * rows_per_sub + k * 128, 128)]
            )
        plsc.subcore_barrier()

        bufs = (rows0, rows1)
        gsems = (gsem0, gsem1)
        ssems = (ssem0, ssem1)
        nhalf = stage_rows // 2

        for s in range(n_stages):
            base = base_row + s * stage_rows

            pltpu.sync_copy(et_hbm.at[pl.ds(base, stage_rows)], dstb)

            def gidx1_body(c, _):
                for j in range(128 // 16):
                    gidxb[c, pl.ds(j * 16, 16)] = (
                        dstb[c, pl.ds(j * 16, 16)] * n_nodes
                    )
                return 0

            lax.fori_loop(0, stage_rows, gidx1_body, 0)
            pltpu.sync_copy(src_hbm.at[pl.ds(base, stage_rows)], dstb)

            def gidx2_body(c, _):
                for j in range(128 // 16):
                    gidxb[c, pl.ds(j * 16, 16)] = (
                        gidxb[c, pl.ds(j * 16, 16)] + dstb[c, pl.ds(j * 16, 16)]
                    )
                return 0

            lax.fori_loop(0, stage_rows, gidx2_body, 0)
            pltpu.sync_copy(dst_hbm.at[pl.ds(base, stage_rows)], dstb)

            pltpu.async_copy(t_hbm.at[gidxb.at[0]], rows0, gsem0)
            pltpu.async_copy(t_hbm.at[gidxb.at[1]], rows1, gsem1)

            def chunk_body(i, _):
                scat = []
                for p in range(2):
                    k = 2 * i + p
                    pltpu.make_async_copy(
                        t_hbm.at[gidxb.at[k]], bufs[p], gsems[p]
                    ).wait()
                    scat.append(pltpu.async_copy(
                        bufs[p], acc_sh.at[dstb.at[k]], ssems[p], add=True
                    ))
                for p in range(2):
                    scat[p].wait()

                    @pl.when(i < nhalf - 1)
                    def _(p=p):
                        pltpu.async_copy(
                            t_hbm.at[gidxb.at[2 * i + 2 + p]], bufs[p], gsems[p]
                        )
                return 0

            lax.fori_loop(0, nhalf, chunk_body, 0)
        plsc.subcore_barrier()

        pltpu.sync_copy(
            acc_sh.at[pl.ds(sid * rows_per_sub, rows_per_sub)],
            acc_out.at[cid, pl.ds(sid * rows_per_sub, rows_per_sub)],
        )

    return pl.kernel(
        body,
        mesh=_sc_mesh(),
        compiler_params=pltpu.CompilerParams(needs_layout_passes=False, use_tc_tiling_on_sc=False),
        out_type=[jax.ShapeDtypeStruct((_NC, n_pad, h), jnp.bfloat16)],
        scratch_types=[
            pltpu.VMEM((rows_per_tile // 2, 128), jnp.int32),
            pltpu.VMEM((rows_per_tile // 2, 128), jnp.int32),
            pltpu.VMEM((128, h), jnp.bfloat16),
            pltpu.VMEM((128, h), jnp.bfloat16),
            pltpu.VMEM_SHARED((n_pad, h), jnp.bfloat16),
            pltpu.SemaphoreType.DMA,
            pltpu.SemaphoreType.DMA,
            pltpu.SemaphoreType.DMA,
            pltpu.SemaphoreType.DMA,
        ],
    )


def kernel(node_feats, edge_index, etype, V1, a1, loop1, b1, V2, a2, loop2, b2):
    n, h = node_feats.shape
    num_bases = V1.shape[0]
    num_rels = a1.shape[0]
    e = etype.shape[0]
    nw = _NC * _NS

    n_pad = ((n + _NS * 128 - 1) // (_NS * 128)) * (_NS * 128)
    rows_per_tile = ((e + nw * 128 - 1) // (nw * 128) + 7) // 8 * 8
    e_pad = rows_per_tile * 128 * nw

    src = edge_index[0].astype(jnp.int32)
    dst = edge_index[1].astype(jnp.int32)
    et = etype.astype(jnp.int32)

    npad = e_pad - e
    pad_i = jnp.arange(npad, dtype=jnp.int32)
    src_p = jnp.concatenate([src, (pad_i * 7919) % n])
    et_p = jnp.concatenate([et, jnp.zeros((npad,), jnp.int32)])
    dst_p = jnp.concatenate([dst, n + (pad_i % (n_pad - n))])

    src2 = src_p.reshape(nw * rows_per_tile, 128)
    et2 = et_p.reshape(nw * rows_per_tile, 128)
    dst2 = dst_p.reshape(nw * rows_per_tile, 128)

    wstack_fn = _make_wstack(num_rels, num_bases, h)
    mm_fn = _make_matmul(n, h, num_rels, 1000)
    deg_fn = _make_sc_deg(n_pad, rows_per_tile)
    agg_fn = _make_sc_agg(n_pad, rows_per_tile, h, n)
    fin_relu = _make_finish(n, n_pad, h, 1000, True)
    fin_last = _make_finish(n, n_pad, h, 1000, False)

    b1r = b1.reshape(1, h)
    b2r = b2.reshape(1, h)

    (degp,) = deg_fn(dst2)
    degp = degp.reshape(_NC, n_pad, 1)

    t1 = mm_fn(node_feats, wstack_fn(a1, V1)).reshape(num_rels * n, h)
    (accp1,) = agg_fn(src2, et2, dst2, t1)
    out1 = fin_relu(accp1, degp, node_feats, loop1, b1r)

    t2 = mm_fn(out1, wstack_fn(a2, V2)).reshape(num_rels * n, h)
    (accp2,) = agg_fn(src2, et2, dst2, t2)
    out2 = fin_last(accp2, degp, out1, loop2, b2r)
    return out2


# speedup vs baseline: 1.1701x; 1.1701x over previous
import functools

import jax
import jax.numpy as jnp
from jax import lax
from jax.experimental import pallas as pl
from jax.experimental.pallas import tpu as pltpu
from jax.experimental.pallas import tpu_sc as plsc

_NC, _NS = 2, 16


def _sc_mesh():
    return plsc.VectorSubcoreMesh(
        core_axis_name="c", subcore_axis_name="s", num_cores=_NC,
        num_subcores=_NS,
    )


@functools.lru_cache(maxsize=None)
def _make_wstack(num_rels: int, num_bases: int, h: int):
    def body(a_ref, v_ref, out_ref):
        r = pl.program_id(0)
        acc = a_ref[r, 0] * v_ref[0]
        for b in range(1, num_bases):
            acc = acc + a_ref[r, b] * v_ref[b]
        out_ref[...] = acc

    return pl.pallas_call(
        body,
        grid=(num_rels,),
        in_specs=[
            pl.BlockSpec((num_rels, num_bases), lambda r: (0, 0)),
            pl.BlockSpec((num_bases, h, h), lambda r: (0, 0, 0)),
        ],
        out_specs=pl.BlockSpec((h, h), lambda r: (0, r)),
        out_shape=jax.ShapeDtypeStruct((h, num_rels * h), jnp.float32),
    )


@functools.lru_cache(maxsize=None)
def _make_matmul(n: int, h: int, num_rels: int, bn: int):
    nb = n // bn

    def body(x_ref, w_ref, out_ref):
        y = jnp.dot(
            x_ref[...].astype(jnp.bfloat16),
            w_ref[...].astype(jnp.bfloat16),
            preferred_element_type=jnp.float32,
        )
        for r in range(num_rels):
            out_ref[r] = y[:, r * h:(r + 1) * h]

    return pl.pallas_call(
        body,
        grid=(nb,),
        in_specs=[
            pl.BlockSpec((bn, h), lambda i: (i, 0)),
            pl.BlockSpec((h, num_rels * h), lambda i: (0, 0)),
        ],
        out_specs=pl.BlockSpec((num_rels, bn, h), lambda i: (0, i, 0)),
        out_shape=jax.ShapeDtypeStruct((num_rels, n, h), jnp.float32),
    )


@functools.lru_cache(maxsize=None)
def _make_finish(n: int, n_pad: int, h: int, bn: int, relu: bool):
    def body(acc_ref, deg_ref, x_ref, loop_ref, b_ref, out_ref):
        acc = acc_ref[0] + acc_ref[1]
        deg = deg_ref[0] + deg_ref[1]
        norm = jnp.where(deg > 0.0, 1.0 / jnp.maximum(deg, 1.0), 0.0)
        y = acc * norm + jnp.dot(
            x_ref[...], loop_ref[...], preferred_element_type=jnp.float32
        ) + b_ref[...]
        if relu:
            y = jnp.maximum(y, 0.0)
        out_ref[...] = y

    return pl.pallas_call(
        body,
        grid=(n // bn,),
        in_specs=[
            pl.BlockSpec((2, bn, h), lambda i: (0, i, 0)),
            pl.BlockSpec((2, bn, 1), lambda i: (0, i, 0)),
            pl.BlockSpec((bn, h), lambda i: (i, 0)),
            pl.BlockSpec((h, h), lambda i: (0, 0)),
            pl.BlockSpec((1, h), lambda i: (0, 0)),
        ],
        out_specs=pl.BlockSpec((bn, h), lambda i: (i, 0)),
        out_shape=jax.ShapeDtypeStruct((n, h), jnp.float32),
    )


@functools.lru_cache(maxsize=None)
def _make_sc_deg(n_pad: int, rows_per_tile: int):
    nrows = n_pad // 128
    n_writers = nrows // 8

    def body(dst_hbm, deg_out, dstb, degl, iotab, deg_sh):
        cid = lax.axis_index("c")
        sid = lax.axis_index("s")
        wid = cid * _NS + sid

        zeros16 = jnp.zeros((16,), jnp.float32)
        ones16 = jnp.ones((16,), jnp.float32)
        iota16 = lax.iota(jnp.int32, 16)

        def fill_body(i, _):
            for j in range(128 // 16):
                degl[i, pl.ds(j * 16, 16)] = zeros16
            return 0

        lax.fori_loop(0, nrows, fill_body, 0)
        for j in range(nrows // 16):
            iotab[pl.ds(j * 16, 16)] = iota16 + j * 16

        @pl.when(sid < n_writers)
        def _():
            pltpu.sync_copy(
                degl.at[pl.ds(0, 8)], deg_sh.at[pl.ds(sid * 8, 8)]
            )
        plsc.subcore_barrier()

        pltpu.sync_copy(dst_hbm.at[pl.ds(wid * rows_per_tile, rows_per_tile)],
                        dstb)

        def chunk_body(c, _):
            for j in range(128 // 16):
                dvec = dstb[c, pl.ds(j * 16, 16)]
                hi = lax.shift_right_logical(dvec, 7)
                lo = lax.bitwise_and(dvec, 127)
                plsc.addupdate_scatter(degl, [hi, lo], ones16)
            return 0

        lax.fori_loop(0, rows_per_tile, chunk_body, 0)

        pltpu.sync_copy(degl, deg_sh.at[iotab], add=True)
        plsc.subcore_barrier()

        @pl.when(sid < n_writers)
        def _():
            pltpu.sync_copy(
                deg_sh.at[pl.ds(sid * 8, 8)],
                deg_out.at[cid, pl.ds(sid * 8, 8)],
            )

    return pl.kernel(
        body,
        mesh=_sc_mesh(),
        compiler_params=pltpu.CompilerParams(needs_layout_passes=False),
        out_type=[jax.ShapeDtypeStruct((_NC, nrows, 128), jnp.float32)],
        scratch_types=[
            pltpu.VMEM((rows_per_tile, 128), jnp.int32),
            pltpu.VMEM((nrows, 128), jnp.float32),
            pltpu.VMEM((nrows,), jnp.int32),
            pltpu.VMEM_SHARED((nrows, 128), jnp.float32),
        ],
    )


@functools.lru_cache(maxsize=None)
def _make_sc_agg(n_pad: int, rows_per_tile: int, h: int, n_nodes: int):
    rows_per_sub = n_pad // _NS
    n128 = rows_per_sub // 128
    n_stages = 2
    stage_rows = rows_per_tile // n_stages

    def body(src_hbm, et_hbm, dst_hbm, t_hbm, acc_out,
             dstb, gidxb, rows0, rows1, acc_sh,
             gsem0, gsem1, ssem0, ssem1):
        cid = lax.axis_index("c")
        sid = lax.axis_index("s")
        wid = cid * _NS + sid
        base_row = wid * rows_per_tile

        zeros16 = jnp.zeros((16,), jnp.float32)

        def fill_body(i, _):
            for j in range(h // 16):
                rows0[i, pl.ds(j * 16, 16)] = zeros16
            return 0

        lax.fori_loop(0, 128, fill_body, 0)
        for k in range(n128):
            pltpu.sync_copy(
                rows0, acc_sh.at[pl.ds(sid * rows_per_sub + k * 128, 128)]
            )
        plsc.subcore_barrier()

        bufs = (rows0, rows1)
        gsems = (gsem0, gsem1)
        ssems = (ssem0, ssem1)
        nhalf = stage_rows // 2

        for s in range(n_stages):
            base = base_row + s * stage_rows

            pltpu.sync_copy(et_hbm.at[pl.ds(base, stage_rows)], dstb)

            def gidx1_body(c, _):
                for j in range(128 // 16):
                    gidxb[c, pl.ds(j * 16, 16)] = (
                        dstb[c, pl.ds(j * 16, 16)] * n_nodes
                    )
                return 0

            lax.fori_loop(0, stage_rows, gidx1_body, 0)
            pltpu.sync_copy(src_hbm.at[pl.ds(base, stage_rows)], dstb)

            def gidx2_body(c, _):
                for j in range(128 // 16):
                    gidxb[c, pl.ds(j * 16, 16)] = (
                        gidxb[c, pl.ds(j * 16, 16)] + dstb[c, pl.ds(j * 16, 16)]
                    )
                return 0

            lax.fori_loop(0, stage_rows, gidx2_body, 0)
            pltpu.sync_copy(dst_hbm.at[pl.ds(base, stage_rows)], dstb)

            pltpu.async_copy(t_hbm.at[gidxb.at[0]], rows0, gsem0)
            pltpu.async_copy(t_hbm.at[gidxb.at[1]], rows1, gsem1)

            def chunk_body(i, _):
                scat = []
                for p in range(2):
                    k = 2 * i + p
                    pltpu.make_async_copy(
                        t_hbm.at[gidxb.at[k]], bufs[p], gsems[p]
                    ).wait()
                    scat.append(pltpu.async_copy(
                        bufs[p], acc_sh.at[dstb.at[k]], ssems[p], add=True
                    ))
                for p in range(2):
                    scat[p].wait()

                    @pl.when(i < nhalf - 1)
                    def _(p=p):
                        pltpu.async_copy(
                            t_hbm.at[gidxb.at[2 * i + 2 + p]], bufs[p], gsems[p]
                        )
                return 0

            lax.fori_loop(0, nhalf, chunk_body, 0)
        plsc.subcore_barrier()

        pltpu.sync_copy(
            acc_sh.at[pl.ds(sid * rows_per_sub, rows_per_sub)],
            acc_out.at[cid, pl.ds(sid * rows_per_sub, rows_per_sub)],
        )

    return pl.kernel(
        body,
        mesh=_sc_mesh(),
        compiler_params=pltpu.CompilerParams(needs_layout_passes=False),
        out_type=[jax.ShapeDtypeStruct((_NC, n_pad, h), jnp.float32)],
        scratch_types=[
            pltpu.VMEM((rows_per_tile // 2, 128), jnp.int32),
            pltpu.VMEM((rows_per_tile // 2, 128), jnp.int32),
            pltpu.VMEM((128, h), jnp.float32),
            pltpu.VMEM((128, h), jnp.float32),
            pltpu.VMEM_SHARED((n_pad, h), jnp.float32),
            pltpu.SemaphoreType.DMA,
            pltpu.SemaphoreType.DMA,
            pltpu.SemaphoreType.DMA,
            pltpu.SemaphoreType.DMA,
        ],
    )


def kernel(node_feats, edge_index, etype, V1, a1, loop1, b1, V2, a2, loop2, b2):
    n, h = node_feats.shape
    num_bases = V1.shape[0]
    num_rels = a1.shape[0]
    e = etype.shape[0]
    nw = _NC * _NS

    n_pad = ((n + _NS * 128 - 1) // (_NS * 128)) * (_NS * 128)
    rows_per_tile = ((e + nw * 128 - 1) // (nw * 128) + 7) // 8 * 8
    e_pad = rows_per_tile * 128 * nw

    src = edge_index[0].astype(jnp.int32)
    dst = edge_index[1].astype(jnp.int32)
    et = etype.astype(jnp.int32)

    npad = e_pad - e
    pad_i = jnp.arange(npad, dtype=jnp.int32)
    src_p = jnp.concatenate([src, (pad_i * 7919) % n])
    et_p = jnp.concatenate([et, jnp.zeros((npad,), jnp.int32)])
    dst_p = jnp.concatenate([dst, n + (pad_i % (n_pad - n))])

    src2 = src_p.reshape(nw * rows_per_tile, 128)
    et2 = et_p.reshape(nw * rows_per_tile, 128)
    dst2 = dst_p.reshape(nw * rows_per_tile, 128)

    wstack_fn = _make_wstack(num_rels, num_bases, h)
    mm_fn = _make_matmul(n, h, num_rels, 1000)
    deg_fn = _make_sc_deg(n_pad, rows_per_tile)
    agg_fn = _make_sc_agg(n_pad, rows_per_tile, h, n)
    fin_relu = _make_finish(n, n_pad, h, 1000, True)
    fin_last = _make_finish(n, n_pad, h, 1000, False)

    b1r = b1.reshape(1, h)
    b2r = b2.reshape(1, h)

    (degp,) = deg_fn(dst2)
    degp = degp.reshape(_NC, n_pad, 1)

    t1 = mm_fn(node_feats, wstack_fn(a1, V1)).reshape(num_rels * n, h)
    (accp1,) = agg_fn(src2, et2, dst2, t1)
    out1 = fin_relu(accp1, degp, node_feats, loop1, b1r)

    t2 = mm_fn(out1, wstack_fn(a2, V2)).reshape(num_rels * n, h)
    (accp2,) = agg_fn(src2, et2, dst2, t2)
    out2 = fin_last(accp2, degp, out1, loop2, b2r)
    return out2
